# Initial kernel scaffold; baseline (speedup 1.0000x reference)
import jax
import jax.numpy as jnp
from jax.experimental import pallas as pl


def kernel(x, edge_index, Wl0, bl0, Wr0, Wl1, bl1, Wr1, Wl2, bl2, Wr2, W1, b1, W2, b2):
    raise NotImplementedError("write your pallas kernel here")


# speedup vs baseline: 7.2206x; 7.2206x over previous
import functools

import jax
import jax.numpy as jnp
from jax import lax
from jax.experimental import pallas as pl
from jax.experimental.pallas import tpu as pltpu
from jax.experimental.pallas import tpu_sc as plsc

_N = 10000
_E = 320000
_H = 128

_NCORE = 2
_NSUB = 16
_NTILE = _NCORE * _NSUB
_EPW = _E // _NTILE
_K = 80
_NCHUNK = _EPW // _K
_ROWS_PER_SUB = 632
_NPAD = _ROWS_PER_SUB * _NSUB
_CPAD = 640 * _NSUB

_mesh = plsc.VectorSubcoreMesh(core_axis_name="c", subcore_axis_name="s")


def _sc_agg_body(with_cnt, *refs):
    if with_cnt:
        (h_hbm, src_hbm, dst_hbm, zrow_hbm, zcnt_hbm,
         agg_out, cnt_out, src_v, dst_v, rows_v, ones_v, agg_sh, cnt_sh,
         sem) = refs
    else:
        (h_hbm, src_hbm, dst_hbm, zrow_hbm,
         agg_out, src_v, dst_v, rows_v, agg_sh,
         sem) = refs
    c = lax.axis_index("c")
    s = lax.axis_index("s")
    wid = c * _NSUB + s

    pltpu.sync_copy(zrow_hbm, agg_sh.at[pl.ds(s * _ROWS_PER_SUB, _ROWS_PER_SUB)])
    if with_cnt:
        pltpu.sync_copy(zcnt_hbm, cnt_sh.at[pl.ds(s * 640, 640)])
        for j in range(_K // 16):
            ones_v[pl.ds(j * 16, 16)] = jnp.full((16,), 1.0, jnp.float32)

    pltpu.sync_copy(src_hbm.at[wid], src_v)
    pltpu.sync_copy(dst_hbm.at[wid], dst_v)
    plsc.subcore_barrier()

    def chunk(i, carry):
        src_row = src_v.at[i]
        dst_row = dst_v.at[i]
        pltpu.async_copy(h_hbm.at[src_row], rows_v, sem).wait()
        pltpu.sync_copy(rows_v, agg_sh.at[dst_row], add=True)
        if with_cnt:
            pltpu.sync_copy(ones_v, cnt_sh.at[dst_row], add=True)
        return carry

    lax.fori_loop(0, _NCHUNK, chunk, 0)
    plsc.subcore_barrier()

    rs = pl.ds(s * _ROWS_PER_SUB, _ROWS_PER_SUB)
    pltpu.sync_copy(agg_sh.at[rs], agg_out.at[c, rs])
    if with_cnt:
        cs = pl.ds(s * 640, 640)
        pltpu.sync_copy(cnt_sh.at[cs], cnt_out.at[c, cs])


_sc_agg_cnt = functools.partial(
    pl.kernel,
    out_type=(
        jax.ShapeDtypeStruct((_NCORE, _NPAD, _H), jnp.float32),
        jax.ShapeDtypeStruct((_NCORE, _CPAD), jnp.float32),
    ),
    mesh=_mesh,
    scratch_types=[
        pltpu.VMEM((_NCHUNK, _K), jnp.int32),
        pltpu.VMEM((_NCHUNK, _K), jnp.int32),
        pltpu.VMEM((_K, _H), jnp.float32),
        pltpu.VMEM((_K,), jnp.float32),
        pltpu.VMEM_SHARED((_NPAD, _H), jnp.float32),
        pltpu.VMEM_SHARED((_CPAD,), jnp.float32),
        pltpu.SemaphoreType.DMA,
    ],
)(functools.partial(_sc_agg_body, True))

_sc_agg = functools.partial(
    pl.kernel,
    out_type=jax.ShapeDtypeStruct((_NCORE, _NPAD, _H), jnp.float32),
    mesh=_mesh,
    scratch_types=[
        pltpu.VMEM((_NCHUNK, _K), jnp.int32),
        pltpu.VMEM((_NCHUNK, _K), jnp.int32),
        pltpu.VMEM((_K, _H), jnp.float32),
        pltpu.VMEM_SHARED((_NPAD, _H), jnp.float32),
        pltpu.SemaphoreType.DMA,
    ],
)(functools.partial(_sc_agg_body, False))


_BN = 1000


def _tc_layer_body(p0, p1, c0, c1, h, wl, bl, wr, out):
    cnt = c0[...] + c1[...]
    inv = 1.0 / jnp.maximum(cnt, 1.0)
    mean = (p0[...] + p1[...]) * inv
    acc = jnp.dot(mean, wl[...], preferred_element_type=jnp.float32)
    acc += jnp.dot(h[...], wr[...], preferred_element_type=jnp.float32)
    out[...] = jnp.maximum(acc + bl[...], 0.0)


def _tc_layer(p0, p1, c0, c1, h, wl, bl, wr):
    nb = _N // _BN
    big = pl.BlockSpec((_BN, _H), lambda i: (i, 0))
    col = pl.BlockSpec((_BN, 1), lambda i: (i, 0))
    wspec = pl.BlockSpec((_H, _H), lambda i: (0, 0))
    bspec = pl.BlockSpec((1, _H), lambda i: (0, 0))
    return pl.pallas_call(
        _tc_layer_body,
        grid=(nb,),
        in_specs=[big, big, col, col, big, wspec, bspec, wspec],
        out_specs=big,
        out_shape=jax.ShapeDtypeStruct((_N, _H), jnp.float32),
    )(p0, p1, c0, c1, h, wl, bl, wr)


def _tc_layer3_head_body(p0, p1, c0, c1, h, wl, bl, wr, w1, b1, w2, b2, out):
    cnt = c0[...] + c1[...]
    inv = 1.0 / jnp.maximum(cnt, 1.0)
    mean = (p0[...] + p1[...]) * inv
    acc = jnp.dot(mean, wl[...], preferred_element_type=jnp.float32)
    acc += jnp.dot(h[...], wr[...], preferred_element_type=jnp.float32)
    h3 = jnp.maximum(acc + bl[...], 0.0)
    t = jnp.maximum(
        jnp.dot(h3, w1[...], preferred_element_type=jnp.float32) + b1[...], 0.0)
    out[...] = jnp.dot(t, w2[...], preferred_element_type=jnp.float32) + b2[...]


def _tc_layer3_head(p0, p1, c0, c1, h, wl, bl, wr, w1, b1, w2, b2):
    nb = _N // _BN
    big = pl.BlockSpec((_BN, _H), lambda i: (i, 0))
    col = pl.BlockSpec((_BN, 1), lambda i: (i, 0))
    full = lambda a: pl.BlockSpec(a.shape, lambda i: tuple(0 for _ in a.shape))
    return pl.pallas_call(
        _tc_layer3_head_body,
        grid=(nb,),
        in_specs=[big, big, col, col, big, full(wl), full(bl), full(wr),
                  full(w1), full(b1), full(w2), full(b2)],
        out_specs=pl.BlockSpec((_BN, 2), lambda i: (i, 0)),
        out_shape=jax.ShapeDtypeStruct((_N, 2), jnp.float32),
    )(p0, p1, c0, c1, h, wl, bl, wr, w1, b1, w2, b2)


def kernel(x, edge_index, Wl0, bl0, Wr0, Wl1, bl1, Wr1, Wl2, bl2, Wr2,
           W1, b1, W2, b2):
    src = edge_index[0].reshape(_NTILE, _NCHUNK, _K)
    dst = edge_index[1].reshape(_NTILE, _NCHUNK, _K)
    zrow = jnp.zeros((_ROWS_PER_SUB, _H), jnp.float32)
    zcnt = jnp.zeros((640,), jnp.float32)
    bl0r, bl1r, bl2r = (b.reshape(1, -1) for b in (bl0, bl1, bl2))
    b1r = b1.reshape(1, -1)
    b2r = b2.reshape(1, -1)

    agg, cnt = _sc_agg_cnt(x, src, dst, zrow, zcnt)
    c0 = cnt[0, :_N].reshape(_N, 1)
    c1 = cnt[1, :_N].reshape(_N, 1)
    h1 = _tc_layer(agg[0], agg[1], c0, c1, x, Wl0, bl0r, Wr0)
    agg = _sc_agg(h1, src, dst, zrow)
    h2 = _tc_layer(agg[0], agg[1], c0, c1, h1, Wl1, bl1r, Wr1)
    agg = _sc_agg(h2, src, dst, zrow)
    return _tc_layer3_head(agg[0], agg[1], c0, c1, h2, Wl2, bl2r, Wr2,
                           W1, b1r, W2, b2r)
